# baseline (speedup 1.0000x reference)
import jax
import jax.numpy as jnp
from jax.experimental import pallas as pl
from jax.experimental.pallas import tpu as pltpu

_IT = 1000


def _softmax_kernel(alf_ref, gf_ref, p_ref):
    a = jax.nn.sigmoid(alf_ref[0])
    row = gf_ref[...]
    m = jnp.max(row)
    e = jnp.exp(row - m)
    p_ref[...] = (1.0 - a) * (e / jnp.sum(e))


def _blend_kernel(alf_ref, p_ref, his_ref, out_ref):
    a = jax.nn.sigmoid(alf_ref[0])
    g_col = p_ref[0, 0, :].reshape(_IT, 1)
    out_ref[...] = a * his_ref[...] + g_col


def kernel(his, global_freq_table, alf):
    batch, num_items = his.shape
    p = pl.pallas_call(
        _softmax_kernel,
        in_specs=[
            pl.BlockSpec(memory_space=pltpu.SMEM),
            pl.BlockSpec(memory_space=pltpu.VMEM),
        ],
        out_specs=pl.BlockSpec(memory_space=pltpu.VMEM),
        out_shape=jax.ShapeDtypeStruct((1, num_items), jnp.float32),
    )(alf, global_freq_table)

    num_tiles = num_items // _IT
    p3 = p.reshape(num_tiles, 1, _IT)
    his_t = his.T
    out_t = pl.pallas_call(
        _blend_kernel,
        grid=(num_tiles,),
        in_specs=[
            pl.BlockSpec(memory_space=pltpu.SMEM),
            pl.BlockSpec((1, 1, _IT), lambda i: (i, 0, 0)),
            pl.BlockSpec((_IT, batch), lambda i: (i, 0)),
        ],
        out_specs=pl.BlockSpec((_IT, batch), lambda i: (i, 0)),
        out_shape=jax.ShapeDtypeStruct((num_items, batch), his.dtype),
    )(alf, p3, his_t)
    return out_t.T
